# Initial kernel scaffold; baseline (speedup 1.0000x reference)
import functools

import jax
import jax.numpy as jnp
from jax import lax
from jax.experimental import pallas as pl
from jax.experimental.pallas import tpu as pltpu

N_NODES = 50000
D = 128
ROW_BLK = 2500
N_BLKS = N_NODES // ROW_BLK


def _dot_t(x, w):
    return lax.dot_general(x, w, (((1,), (1,)), ((), ())),
                           preferred_element_type=jnp.float32)


def _init_project_body(x_ref, w0_ref, b0_ref, wp_ref, bp_ref, x_out_ref, h_out_ref):
    x = x_ref[...]
    x0 = _dot_t(x, w0_ref[...]) + b0_ref[...]
    x_out_ref[...] = x0
    h_out_ref[...] = jnp.maximum(_dot_t(x0, wp_ref[...]) + bp_ref[...], 0.0)


def _init_project(x, w0, b0, wp, bp):
    return pl.pallas_call(
        _init_project_body,
        grid=(N_BLKS,),
        in_specs=[
            pl.BlockSpec((ROW_BLK, D), lambda i: (i, 0)),
            pl.BlockSpec((D, D), lambda i: (0, 0)),
            pl.BlockSpec((1, D), lambda i: (0, 0)),
            pl.BlockSpec((D, D), lambda i: (0, 0)),
            pl.BlockSpec((1, D), lambda i: (0, 0)),
        ],
        out_specs=[
            pl.BlockSpec((ROW_BLK, D), lambda i: (i, 0)),
            pl.BlockSpec((ROW_BLK, D), lambda i: (i, 0)),
        ],
        out_shape=[
            jax.ShapeDtypeStruct((N_NODES, D), jnp.float32),
            jax.ShapeDtypeStruct((N_NODES, D), jnp.float32),
        ],
    )(x, w0, b0.reshape(1, D), wp, bp.reshape(1, D))


def _sage_update_body(agg_ref, cnt_ref, xd_ref, wl_ref, bl_ref, wr_ref,
                      wp_ref, bp_ref, x_out_ref, h_out_ref):
    mean = agg_ref[...] / jnp.maximum(cnt_ref[...], 1.0)
    out = _dot_t(mean, wl_ref[...]) + bl_ref[...] + _dot_t(xd_ref[...], wr_ref[...])
    norm = jnp.sqrt(jnp.sum(out * out, axis=-1, keepdims=True))
    xn = out / jnp.maximum(norm, 1e-12)
    x_out_ref[...] = xn
    h_out_ref[...] = jnp.maximum(_dot_t(xn, wp_ref[...]) + bp_ref[...], 0.0)


def _sage_update(agg, cnt, x_dst, wl, bl, wr, wp, bp):
    return pl.pallas_call(
        _sage_update_body,
        grid=(N_BLKS,),
        in_specs=[
            pl.BlockSpec((ROW_BLK, D), lambda i: (i, 0)),
            pl.BlockSpec((ROW_BLK, 1), lambda i: (i, 0)),
            pl.BlockSpec((ROW_BLK, D), lambda i: (i, 0)),
            pl.BlockSpec((D, D), lambda i: (0, 0)),
            pl.BlockSpec((1, D), lambda i: (0, 0)),
            pl.BlockSpec((D, D), lambda i: (0, 0)),
            pl.BlockSpec((D, D), lambda i: (0, 0)),
            pl.BlockSpec((1, D), lambda i: (0, 0)),
        ],
        out_specs=[
            pl.BlockSpec((ROW_BLK, D), lambda i: (i, 0)),
            pl.BlockSpec((ROW_BLK, D), lambda i: (i, 0)),
        ],
        out_shape=[
            jax.ShapeDtypeStruct((N_NODES, D), jnp.float32),
            jax.ShapeDtypeStruct((N_NODES, D), jnp.float32),
        ],
    )(agg, cnt.reshape(N_NODES, 1), x_dst, wl, bl.reshape(1, D), wr,
      wp, bp.reshape(1, D))


def _sage_final_body(agg_ref, cnt_ref, xd_ref, wl_ref, bl_ref, wr_ref,
                     w1_ref, b1_ref, w2_ref, b2_ref, y_ref):
    mean = agg_ref[...] / jnp.maximum(cnt_ref[...], 1.0)
    out = _dot_t(mean, wl_ref[...]) + bl_ref[...] + _dot_t(xd_ref[...], wr_ref[...])
    norm = jnp.sqrt(jnp.sum(out * out, axis=-1, keepdims=True))
    xn = out / jnp.maximum(norm, 1e-12)
    h = jnp.maximum(_dot_t(xn, w1_ref[...]) + b1_ref[...], 0.0)
    y_ref[...] = _dot_t(h, w2_ref[...]) + b2_ref[...]


def _sage_final(agg, cnt, x_dst, wl, bl, wr, w1, b1, w2, b2):
    w2p = jnp.zeros((8, D), jnp.float32).at[:2].set(w2)
    b2p = jnp.zeros((1, 8), jnp.float32).at[0, :2].set(b2)
    y = pl.pallas_call(
        _sage_final_body,
        grid=(N_BLKS,),
        in_specs=[
            pl.BlockSpec((ROW_BLK, D), lambda i: (i, 0)),
            pl.BlockSpec((ROW_BLK, 1), lambda i: (i, 0)),
            pl.BlockSpec((ROW_BLK, D), lambda i: (i, 0)),
            pl.BlockSpec((D, D), lambda i: (0, 0)),
            pl.BlockSpec((1, D), lambda i: (0, 0)),
            pl.BlockSpec((D, D), lambda i: (0, 0)),
            pl.BlockSpec((D, D), lambda i: (0, 0)),
            pl.BlockSpec((1, D), lambda i: (0, 0)),
            pl.BlockSpec((8, D), lambda i: (0, 0)),
            pl.BlockSpec((1, 8), lambda i: (0, 0)),
        ],
        out_specs=pl.BlockSpec((ROW_BLK, 8), lambda i: (i, 0)),
        out_shape=jax.ShapeDtypeStruct((N_NODES, 8), jnp.float32),
    )(agg, cnt.reshape(N_NODES, 1), x_dst, wl, bl.reshape(1, D), wr,
      w1, b1.reshape(1, D), w2p, b2p)
    return y[:, :2]


def _aggregate(h, src, dst):
    msg = jnp.take(h, src, axis=0)
    agg = jax.ops.segment_sum(msg, dst, num_segments=N_NODES)
    cnt = jax.ops.segment_sum(jnp.ones((src.shape[0],), jnp.float32), dst,
                              num_segments=N_NODES)
    return agg, cnt


def kernel(x_reactions, x_constraints, edge_index_rc, edge_index_cr, params):
    p = params
    src_rc, dst_rc = edge_index_rc[0], edge_index_rc[1]
    src_cr, dst_cr = edge_index_cr[0], edge_index_cr[1]

    x_r, h_r1 = _init_project(x_reactions, p["init_rec_W"], p["init_rec_b"],
                              p["sage_rc1"]["Wp"], p["sage_rc1"]["bp"])
    x_c, _ = _init_project(x_constraints, p["init_con_W"], p["init_con_b"],
                           p["sage_rc1"]["Wp"], p["sage_rc1"]["bp"])

    agg, cnt_rc = _aggregate(h_r1, src_rc, dst_rc)
    s = p["sage_rc1"]
    x_c, h_c2 = _sage_update(agg, cnt_rc, x_c, s["Wl"], s["bl"], s["Wr"],
                             p["sage_cr1"]["Wp"], p["sage_cr1"]["bp"])

    agg, cnt_cr = _aggregate(h_c2, src_cr, dst_cr)
    s = p["sage_cr1"]
    x_r, h_r2 = _sage_update(agg, cnt_cr, x_r, s["Wl"], s["bl"], s["Wr"],
                             p["sage_rc2"]["Wp"], p["sage_rc2"]["bp"])

    agg, _ = _aggregate(h_r2, src_rc, dst_rc)
    s = p["sage_rc2"]
    x_c, h_c3 = _sage_update(agg, cnt_rc, x_c, s["Wl"], s["bl"], s["Wr"],
                             p["sage_cr2"]["Wp"], p["sage_cr2"]["bp"])

    agg, _ = _aggregate(h_c3, src_cr, dst_cr)
    s = p["sage_cr2"]
    return _sage_final(agg, cnt_cr, x_r, s["Wl"], s["bl"], s["Wr"],
                       p["out1_W"], p["out1_b"], p["out2_W"], p["out2_b"])


# speedup vs baseline: 1.0084x; 1.0084x over previous
import functools

import jax
import jax.numpy as jnp
from jax import lax
from jax.experimental import pallas as pl
from jax.experimental.pallas import tpu as pltpu

N_NODES = 50000
D = 128
ROW_BLK = 2000
N_BLKS = N_NODES // ROW_BLK


def _dot_t(x, w):
    return lax.dot_general(x, w, (((1,), (1,)), ((), ())),
                           preferred_element_type=jnp.float32)


def _init_project_body(x_ref, w0_ref, b0_ref, wp_ref, bp_ref, x_out_ref, h_out_ref):
    x = x_ref[...]
    x0 = _dot_t(x, w0_ref[...]) + b0_ref[...]
    x_out_ref[...] = x0
    h_out_ref[...] = jnp.maximum(_dot_t(x0, wp_ref[...]) + bp_ref[...], 0.0)


def _init_project(x, w0, b0, wp, bp):
    return pl.pallas_call(
        _init_project_body,
        grid=(N_BLKS,),
        in_specs=[
            pl.BlockSpec((ROW_BLK, D), lambda i: (i, 0)),
            pl.BlockSpec((D, D), lambda i: (0, 0)),
            pl.BlockSpec((1, D), lambda i: (0, 0)),
            pl.BlockSpec((D, D), lambda i: (0, 0)),
            pl.BlockSpec((1, D), lambda i: (0, 0)),
        ],
        out_specs=[
            pl.BlockSpec((ROW_BLK, D), lambda i: (i, 0)),
            pl.BlockSpec((ROW_BLK, D), lambda i: (i, 0)),
        ],
        out_shape=[
            jax.ShapeDtypeStruct((N_NODES, D), jnp.float32),
            jax.ShapeDtypeStruct((N_NODES, D), jnp.float32),
        ],
    )(x, w0, b0.reshape(1, D), wp, bp.reshape(1, D))


def _sage_update_body(agg_ref, cnt_ref, xd_ref, wl_ref, bl_ref, wr_ref,
                      wp_ref, bp_ref, x_out_ref, h_out_ref):
    mean = agg_ref[...] / jnp.maximum(cnt_ref[...], 1.0)
    out = _dot_t(mean, wl_ref[...]) + bl_ref[...] + _dot_t(xd_ref[...], wr_ref[...])
    norm = jnp.sqrt(jnp.sum(out * out, axis=-1, keepdims=True))
    xn = out / jnp.maximum(norm, 1e-12)
    x_out_ref[...] = xn
    h_out_ref[...] = jnp.maximum(_dot_t(xn, wp_ref[...]) + bp_ref[...], 0.0)


def _sage_update(agg, cnt, x_dst, wl, bl, wr, wp, bp):
    return pl.pallas_call(
        _sage_update_body,
        grid=(N_BLKS,),
        in_specs=[
            pl.BlockSpec((ROW_BLK, D), lambda i: (i, 0)),
            pl.BlockSpec((ROW_BLK, 1), lambda i: (i, 0)),
            pl.BlockSpec((ROW_BLK, D), lambda i: (i, 0)),
            pl.BlockSpec((D, D), lambda i: (0, 0)),
            pl.BlockSpec((1, D), lambda i: (0, 0)),
            pl.BlockSpec((D, D), lambda i: (0, 0)),
            pl.BlockSpec((D, D), lambda i: (0, 0)),
            pl.BlockSpec((1, D), lambda i: (0, 0)),
        ],
        out_specs=[
            pl.BlockSpec((ROW_BLK, D), lambda i: (i, 0)),
            pl.BlockSpec((ROW_BLK, D), lambda i: (i, 0)),
        ],
        out_shape=[
            jax.ShapeDtypeStruct((N_NODES, D), jnp.float32),
            jax.ShapeDtypeStruct((N_NODES, D), jnp.float32),
        ],
    )(agg, cnt.reshape(N_NODES, 1), x_dst, wl, bl.reshape(1, D), wr,
      wp, bp.reshape(1, D))


def _sage_final_body(agg_ref, cnt_ref, xd_ref, wl_ref, bl_ref, wr_ref,
                     w1_ref, b1_ref, w2_ref, b2_ref, y_ref):
    mean = agg_ref[...] / jnp.maximum(cnt_ref[...], 1.0)
    out = _dot_t(mean, wl_ref[...]) + bl_ref[...] + _dot_t(xd_ref[...], wr_ref[...])
    norm = jnp.sqrt(jnp.sum(out * out, axis=-1, keepdims=True))
    xn = out / jnp.maximum(norm, 1e-12)
    h = jnp.maximum(_dot_t(xn, w1_ref[...]) + b1_ref[...], 0.0)
    y_ref[...] = _dot_t(h, w2_ref[...]) + b2_ref[...]


def _sage_final(agg, cnt, x_dst, wl, bl, wr, w1, b1, w2, b2):
    w2p = jnp.zeros((8, D), jnp.float32).at[:2].set(w2)
    b2p = jnp.zeros((1, 8), jnp.float32).at[0, :2].set(b2)
    y = pl.pallas_call(
        _sage_final_body,
        grid=(N_BLKS,),
        in_specs=[
            pl.BlockSpec((ROW_BLK, D), lambda i: (i, 0)),
            pl.BlockSpec((ROW_BLK, 1), lambda i: (i, 0)),
            pl.BlockSpec((ROW_BLK, D), lambda i: (i, 0)),
            pl.BlockSpec((D, D), lambda i: (0, 0)),
            pl.BlockSpec((1, D), lambda i: (0, 0)),
            pl.BlockSpec((D, D), lambda i: (0, 0)),
            pl.BlockSpec((D, D), lambda i: (0, 0)),
            pl.BlockSpec((1, D), lambda i: (0, 0)),
            pl.BlockSpec((8, D), lambda i: (0, 0)),
            pl.BlockSpec((1, 8), lambda i: (0, 0)),
        ],
        out_specs=pl.BlockSpec((ROW_BLK, 8), lambda i: (i, 0)),
        out_shape=jax.ShapeDtypeStruct((N_NODES, 8), jnp.float32),
    )(agg, cnt.reshape(N_NODES, 1), x_dst, wl, bl.reshape(1, D), wr,
      w1, b1.reshape(1, D), w2p, b2p)
    return y[:, :2]


def _aggregate(h, src, dst):
    msg = jnp.take(h, src, axis=0)
    agg = jax.ops.segment_sum(msg, dst, num_segments=N_NODES)
    cnt = jax.ops.segment_sum(jnp.ones((src.shape[0],), jnp.float32), dst,
                              num_segments=N_NODES)
    return agg, cnt


def kernel(x_reactions, x_constraints, edge_index_rc, edge_index_cr, params):
    p = params
    src_rc, dst_rc = edge_index_rc[0], edge_index_rc[1]
    src_cr, dst_cr = edge_index_cr[0], edge_index_cr[1]

    x_r, h_r1 = _init_project(x_reactions, p["init_rec_W"], p["init_rec_b"],
                              p["sage_rc1"]["Wp"], p["sage_rc1"]["bp"])
    x_c, _ = _init_project(x_constraints, p["init_con_W"], p["init_con_b"],
                           p["sage_rc1"]["Wp"], p["sage_rc1"]["bp"])

    agg, cnt_rc = _aggregate(h_r1, src_rc, dst_rc)
    s = p["sage_rc1"]
    x_c, h_c2 = _sage_update(agg, cnt_rc, x_c, s["Wl"], s["bl"], s["Wr"],
                             p["sage_cr1"]["Wp"], p["sage_cr1"]["bp"])

    agg, cnt_cr = _aggregate(h_c2, src_cr, dst_cr)
    s = p["sage_cr1"]
    x_r, h_r2 = _sage_update(agg, cnt_cr, x_r, s["Wl"], s["bl"], s["Wr"],
                             p["sage_rc2"]["Wp"], p["sage_rc2"]["bp"])

    agg, _ = _aggregate(h_r2, src_rc, dst_rc)
    s = p["sage_rc2"]
    x_c, h_c3 = _sage_update(agg, cnt_rc, x_c, s["Wl"], s["bl"], s["Wr"],
                             p["sage_cr2"]["Wp"], p["sage_cr2"]["bp"])

    agg, _ = _aggregate(h_c3, src_cr, dst_cr)
    s = p["sage_cr2"]
    return _sage_final(agg, cnt_cr, x_r, s["Wl"], s["bl"], s["Wr"],
                       p["out1_W"], p["out1_b"], p["out2_W"], p["out2_b"])
